# baseline (speedup 1.0000x reference)
import jax
import jax.numpy as jnp
import numpy as np
from jax import lax
from jax.experimental import pallas as pl
from jax.experimental.pallas import tpu as pltpu
from jax.experimental.pallas import tpu_sc as plsc

L = 50
K = 2000
E = 64
B = 128
SLOTS = 25 * L
DP = 128
CHUNK = 40
NCHUNK = 5
H1 = 256
H2 = 128
NB = 16


def _pair_tables():
    r = np.arange(25)[:, None]
    c = np.arange(L)[None, :]
    first = c < r + 1
    iof = np.where(first, r + 1, 49 - r).reshape(-1)
    jof = np.where(first, c, c - (r + 1)).reshape(-1)
    valid = ((r < 24) | (c < 25)).reshape(-1)
    slots = np.arange(SLOTS)
    fe = np.zeros((SLOTS, 2 * L), np.float32)
    fe[slots[valid], iof[valid]] = 1.0
    fe[slots[valid], L + jof[valid]] = 1.0
    tri = np.zeros((L, SLOTS), np.float32)
    tri[iof[valid], slots[valid]] = 1.0
    return fe, tri


_FE, _TRI = _pair_tables()


def _sc_gather_body(table_hbm, idx_hbm, out_hbm, idx_v, rows_v, sem):
    wid = lax.axis_index("s") * 2 + lax.axis_index("c")
    pltpu.sync_copy(idx_hbm.at[wid], idx_v)
    copies = [
        pltpu.async_copy(table_hbm.at[idx_v.at[c]],
                         rows_v.at[pl.ds(c * CHUNK, CHUNK)], sem)
        for c in range(NCHUNK)
    ]
    for cp in copies:
        cp.wait()
    pltpu.sync_copy(rows_v, out_hbm.at[pl.ds(wid * (CHUNK * NCHUNK),
                                             CHUNK * NCHUNK)])


def _sc_gather(table, idx):
    mesh = plsc.VectorSubcoreMesh(core_axis_name="c", subcore_axis_name="s")
    f = pl.kernel(
        _sc_gather_body,
        mesh=mesh,
        out_type=jax.ShapeDtypeStruct((B * L, DP), jnp.float32),
        scratch_types=[
            pltpu.VMEM((NCHUNK, CHUNK), jnp.int32),
            pltpu.VMEM((CHUNK * NCHUNK, DP), jnp.float32),
            pltpu.SemaphoreType.DMA,
        ],
    )
    return f(table, idx)


def _t0_body(ts_ref, sk_ref, tg_ref, fe_ref, oh_ref, tgj_ref):
    f32 = jnp.float32
    for ib in range(NB):
        trow = ts_ref[ib]
        padrow = (sk_ref[ib] == 0.0).astype(f32)
        zrow = jnp.zeros_like(trow)
        m3 = jnp.concatenate(
            [jnp.concatenate([trow, -trow], axis=1),
             jnp.concatenate([padrow, padrow], axis=1),
             jnp.concatenate([zrow, tg_ref[ib]], axis=1)], axis=0)
        e3 = jnp.dot(m3, fe_ref[...], preferred_element_type=f32)
        dt = e3[0:1, :]
        padsum = e3[1:2, :]
        cat = (1.0 + (dt > 1.0).astype(f32) + (dt > 3600.0).astype(f32)
               + (dt > 86400.0).astype(f32) + (dt > 604800.0).astype(f32))
        cat = jnp.where(padsum > 0.0, 0.0, cat)
        oh_ref[ib] = (cat.astype(jnp.int32)
                      == lax.broadcasted_iota(jnp.int32, (8, SLOTS), 0)
                      ).astype(jnp.bfloat16)
        tgj_ref[ib] = e3[2:3, :]


def _t0_call(interpret, ts3, sk3, tg3, fe):
    def perb(shape):
        return pl.BlockSpec((NB,) + shape[1:], lambda i: (i, 0, 0))

    return pl.pallas_call(
        _t0_body,
        grid=(B // NB,),
        in_specs=[perb(ts3.shape), perb(sk3.shape), perb(tg3.shape),
                  pl.BlockSpec(fe.shape, lambda i: (0, 0))],
        out_specs=[perb((B, 8, SLOTS)), perb((B, 1, SLOTS))],
        out_shape=[jax.ShapeDtypeStruct((B, 8, SLOTS), jnp.bfloat16),
                   jax.ShapeDtypeStruct((B, 1, SLOTS), jnp.float32)],
        interpret=interpret,
    )(ts3, sk3, tg3, fe)


def _tc_body(se_ref, sk_ref, tg_ref, mk_ref, us_ref, it_ref, lg_ref,
             oh_ref, tgj_ref,
             w1a_ref, w1b_ref, c8_ref, w2_ref, b2_ref, wsc_ref,
             bs_ref, wu_ref, wi_ref, wl_ref, blin_ref, tri_ref,
             febf_ref, i3_ref, loss_ref, sig_ref, lab_ref):
    f32 = jnp.float32
    bf = jnp.bfloat16

    def dott(a, b):
        return lax.dot_general(a, b, (((1,), (1,)), ((), ())),
                               preferred_element_type=f32)

    def one(ib):
        ge = se_ref[ib]
        set_ = ge[:, :E]
        pt = dott(w1a_ref[...], set_)
        qt = dott(w1b_ref[...], set_)
        g3 = dott(i3_ref[...], ge[:, E:E + 3])
        skrow = sk_ref[ib]
        tgrow = tg_ref[ib]
        padrow = (skrow == 0.0).astype(f32)
        tgj = tgj_ref[ib]

        aall = jnp.concatenate([pt, qt, c8_ref[...]],
                               axis=1).astype(bf)
        lhs = jnp.concatenate([febf_ref[...], oh_ref[ib]], axis=0)
        h1 = jnp.maximum(
            jnp.dot(aall, lhs, preferred_element_type=f32).astype(bf),
            0.0)
        h2 = jnp.maximum(
            jnp.dot(w2_ref[...], h1, preferred_element_type=f32)
            + b2_ref[...], 0.0)
        s = jnp.tanh(
            jnp.dot(wsc_ref[...], h2, preferred_element_type=f32)
            + bs_ref[0, 0])

        padf = 1.0 - padrow
        sv = jnp.concatenate([s, s * tgj], axis=0)
        vals = jnp.dot(sv, tri_ref[...], preferred_element_type=f32)
        vals1 = vals[0:1, :] * padf
        vals2 = vals[1:2, :] * padf

        udot = jnp.sum(us_ref[ib] * wu_ref[...])
        itdot = dott(wi_ref[...], it_ref[ib])
        ldot = dott(wl_ref[...], lg_ref[ib])
        logits = (udot + itdot + ldot + g3[0:1, :]
                  + g3[1:2, :] * vals1 + g3[2:3, :] * vals2
                  + blin_ref[0, 0])
        m = mk_ref[ib]
        preds = logits * m
        labels = tgrow * m
        loss_ref[ib] = (jnp.maximum(preds, 0.0) - preds * labels
                        + jnp.log1p(jnp.exp(-jnp.abs(preds))))
        sig_ref[ib] = 1.0 / (1.0 + jnp.exp(-preds))
        lab_ref[ib] = labels

    for ib in range(NB):
        one(ib)


def _tc_call(interpret, se3, ts3, sk3, tg3, mk3, us3, it3, lg3,
             w1a, w1b, c8, w2t, b2r, wsc, bsr, wu, wi, wl, blinr,
             fe, tri, febf, i3, oh_all, tgj_all):
    def perb(shape):
        return pl.BlockSpec((NB,) + shape[1:], lambda i: (i, 0, 0))

    def const(arr):
        return pl.BlockSpec(arr.shape, lambda i: (0,) * arr.ndim)

    in_specs = [perb(se3.shape), perb(sk3.shape),
                perb(tg3.shape), perb(mk3.shape), perb(us3.shape),
                perb(it3.shape), perb(lg3.shape),
                perb(oh_all.shape), perb(tgj_all.shape),
                const(w1a), const(w1b), const(c8), const(w2t),
                const(b2r), const(wsc), const(bsr), const(wu), const(wi),
                const(wl), const(blinr), const(tri), const(febf),
                const(i3)]
    out_specs = [perb((B, 1, L))] * 3
    out_shape = [jax.ShapeDtypeStruct((B, 1, L), jnp.float32)] * 3
    return pl.pallas_call(
        _tc_body,
        grid=(B // NB,),
        in_specs=in_specs,
        out_specs=out_specs,
        out_shape=out_shape,
        interpret=interpret,
    )(se3, sk3, tg3, mk3, us3, it3, lg3, oh_all, tgj_all,
      w1a, w1b, c8, w2t, b2r, wsc, bsr, wu, wi, wl, blinr, tri, febf, i3)


def _prep(users, items, langs, skills, timestamps, targets, mask, W1, b1,
          W2, b2, Ws, bs, blin, g):
    set3 = g.reshape(B, L, DP)
    w1a = W1[:, :E]
    w1b = W1[:, E:2 * E]
    c8 = (jnp.concatenate(
        [W1[:, 2 * E:2 * E + 6], jnp.zeros((H1, 2), jnp.float32)], axis=1)
        + b1[:, None])
    w2 = W2.astype(jnp.bfloat16)
    wsc = Ws
    b2r = b2.reshape(H2, 1)
    bsr = bs.reshape(1, 1)
    ts3 = timestamps.astype(jnp.float32).reshape(B, 1, L)
    sk3 = skills.astype(jnp.float32).reshape(B, 1, L)
    tg3 = targets.reshape(B, 1, L)
    mk3 = jnp.asarray(mask).astype(jnp.float32).reshape(B, 1, L)
    us3 = users.reshape(B, 1, 32)
    it3 = items.reshape(B, L, 32)
    lg3 = langs.reshape(B, L, 16)
    return (set3, ts3, sk3, tg3, mk3, us3, it3, lg3,
            w1a, w1b, c8, w2, b2r, wsc, bsr)


def _wlin_split(Wlin):
    w = Wlin[0]
    wu = w[:32].reshape(1, 32)
    wi = w[32:64].reshape(1, 32)
    wl = w[64:80].reshape(1, 16)
    wk = w[80:80 + K]
    wt1 = w[80 + K:80 + 2 * K]
    wt2 = w[80 + 2 * K:80 + 3 * K]
    return wu, wi, wl, wk, wt1, wt2


def kernel(users, items, langs, skills, timestamps, targets, mask, emb_table,
           W1, b1, W2, b2, Ws, bs, Wlin, blin):
    wu, wi, wl, wk, wt1, wt2 = _wlin_split(Wlin)
    table = jnp.concatenate(
        [emb_table, wk[:, None], wt1[:, None], wt2[:, None],
         jnp.zeros((K, DP - E - 3), jnp.float32)], axis=1)
    idx = skills.reshape(-1).astype(jnp.int32).reshape(
        (B * L) // (CHUNK * NCHUNK), NCHUNK, CHUNK)
    g = _sc_gather(table, idx)
    pre = _prep(users, items, langs, skills, timestamps, targets, mask,
                W1, b1, W2, b2, Ws, bs, blin, g)
    blinr = blin.reshape(1, 1)
    fet = jnp.asarray(_FE.T.copy())
    trit = jnp.asarray(_TRI.T.copy())
    i3 = jnp.eye(3, dtype=jnp.float32)
    oh_all, tgj_all = _t0_call(False, pre[1], pre[2], pre[3], fet)
    loss3, sig3, lab3 = _tc_call(False, *pre, wu, wi, wl, blinr,
                                 fet, trit, fet.astype(jnp.bfloat16), i3,
                                 oh_all, tgj_all)
    return (loss3.reshape(-1), sig3.reshape(-1), lab3.reshape(-1))


# speedup vs baseline: 1.0378x; 1.0378x over previous
import jax
import jax.numpy as jnp
import numpy as np
from jax import lax
from jax.experimental import pallas as pl
from jax.experimental.pallas import tpu as pltpu
from jax.experimental.pallas import tpu_sc as plsc

L = 50
K = 2000
E = 64
B = 128
SLOTS = 25 * L
DP = 128
CHUNK = 40
NCHUNK = 5
H1 = 256
H2 = 128
NB = 16


def _pair_tables():
    r = np.arange(25)[:, None]
    c = np.arange(L)[None, :]
    first = c < r + 1
    iof = np.where(first, r + 1, 49 - r).reshape(-1)
    jof = np.where(first, c, c - (r + 1)).reshape(-1)
    valid = ((r < 24) | (c < 25)).reshape(-1)
    slots = np.arange(SLOTS)
    fe = np.zeros((SLOTS, 2 * L), np.float32)
    fe[slots[valid], iof[valid]] = 1.0
    fe[slots[valid], L + jof[valid]] = 1.0
    tri = np.zeros((L, SLOTS), np.float32)
    tri[iof[valid], slots[valid]] = 1.0
    return fe, tri


_FE, _TRI = _pair_tables()


def _sc_gather_body(table_hbm, idx_hbm, out_hbm, idx_v, rows_v, sem):
    wid = lax.axis_index("s") * 2 + lax.axis_index("c")
    pltpu.sync_copy(idx_hbm.at[wid], idx_v)
    copies = [
        pltpu.async_copy(table_hbm.at[idx_v.at[c]],
                         rows_v.at[pl.ds(c * CHUNK, CHUNK)], sem)
        for c in range(NCHUNK)
    ]
    for cp in copies:
        cp.wait()
    pltpu.sync_copy(rows_v, out_hbm.at[pl.ds(wid * (CHUNK * NCHUNK),
                                             CHUNK * NCHUNK)])


def _sc_gather(table, idx):
    mesh = plsc.VectorSubcoreMesh(core_axis_name="c", subcore_axis_name="s")
    f = pl.kernel(
        _sc_gather_body,
        mesh=mesh,
        out_type=jax.ShapeDtypeStruct((B * L, DP), jnp.float32),
        scratch_types=[
            pltpu.VMEM((NCHUNK, CHUNK), jnp.int32),
            pltpu.VMEM((CHUNK * NCHUNK, DP), jnp.float32),
            pltpu.SemaphoreType.DMA,
        ],
    )
    return f(table, idx)


def _tc_body(se_ref, ts_ref, sk_ref, tg_ref, mk_ref, us_ref, it_ref, lg_ref,
             w1a_ref, w1b_ref, c8_ref, w2_ref, b2_ref, wsc_ref,
             bs_ref, wu_ref, wi_ref, wl_ref, blin_ref, fe_ref, tri_ref,
             febf_ref, i3_ref, loss_ref, sig_ref, lab_ref):
    f32 = jnp.float32
    bf = jnp.bfloat16

    def dott(a, b):
        return lax.dot_general(a, b, (((1,), (1,)), ((), ())),
                               preferred_element_type=f32)

    def one(ib):
        ge = se_ref[ib]
        set_ = ge[:, :E].astype(bf)
        pt = dott(w1a_ref[...], set_)
        qt = dott(w1b_ref[...], set_)
        g3 = dott(i3_ref[...], ge[:, E:E + 3])
        trow = ts_ref[ib]
        skrow = sk_ref[ib]
        tgrow = tg_ref[ib]
        padrow = (skrow == 0.0).astype(f32)

        zrow = jnp.zeros_like(trow)
        m3 = jnp.concatenate(
            [jnp.concatenate([trow, -trow], axis=1),
             jnp.concatenate([padrow, padrow], axis=1),
             jnp.concatenate([zrow, tgrow], axis=1)], axis=0)
        e3 = jnp.dot(m3, fe_ref[...], preferred_element_type=f32)
        dt = e3[0:1, :]
        padsum = e3[1:2, :]
        tgj = e3[2:3, :]

        cat = (1.0 + (dt > 1.0).astype(f32) + (dt > 3600.0).astype(f32)
               + (dt > 86400.0).astype(f32) + (dt > 604800.0).astype(f32))
        cat = jnp.where(padsum > 0.0, 0.0, cat)
        oh = (cat.astype(jnp.int32)
              == lax.broadcasted_iota(jnp.int32, (8, SLOTS), 0))

        aall = jnp.concatenate([pt, qt, c8_ref[...]],
                               axis=1).astype(bf)
        lhs = jnp.concatenate([febf_ref[...], oh.astype(bf)], axis=0)
        h1 = jnp.maximum(
            jnp.dot(aall, lhs, preferred_element_type=f32).astype(bf),
            0.0)
        h2 = jnp.maximum(
            jnp.dot(w2_ref[...], h1, preferred_element_type=f32)
            + b2_ref[...], 0.0)
        s = jnp.tanh(
            jnp.dot(wsc_ref[...], h2, preferred_element_type=f32)
            + bs_ref[0, 0])

        padf = 1.0 - padrow
        sv = jnp.concatenate([s, s * tgj], axis=0).astype(bf)
        vals = jnp.dot(sv, tri_ref[...], preferred_element_type=f32)
        vals1 = vals[0:1, :] * padf
        vals2 = vals[1:2, :] * padf

        udot = jnp.sum(us_ref[ib] * wu_ref[...])
        itdot = dott(wi_ref[...], it_ref[ib])
        ldot = dott(wl_ref[...], lg_ref[ib])
        logits = (udot + itdot + ldot + g3[0:1, :]
                  + g3[1:2, :] * vals1 + g3[2:3, :] * vals2
                  + blin_ref[0, 0])
        m = mk_ref[ib]
        preds = logits * m
        labels = tgrow * m
        loss_ref[ib] = (jnp.maximum(preds, 0.0) - preds * labels
                        + jnp.log1p(jnp.exp(-jnp.abs(preds))))
        sig_ref[ib] = 1.0 / (1.0 + jnp.exp(-preds))
        lab_ref[ib] = labels

    for ib in range(NB):
        one(ib)


def _tc_call(interpret, se3, ts3, sk3, tg3, mk3, us3, it3, lg3,
             w1a, w1b, c8, w2t, b2r, wsc, bsr, wu, wi, wl, blinr,
             fe, tri, febf, i3):
    def perb(shape):
        return pl.BlockSpec((NB,) + shape[1:], lambda i: (i, 0, 0))

    def const(arr):
        return pl.BlockSpec(arr.shape, lambda i: (0,) * arr.ndim)

    in_specs = [perb(se3.shape), perb(ts3.shape), perb(sk3.shape),
                perb(tg3.shape), perb(mk3.shape), perb(us3.shape),
                perb(it3.shape), perb(lg3.shape),
                const(w1a), const(w1b), const(c8), const(w2t),
                const(b2r), const(wsc), const(bsr), const(wu), const(wi),
                const(wl), const(blinr), const(fe), const(tri), const(febf),
                const(i3)]
    out_specs = [perb((B, 1, L))] * 3
    out_shape = [jax.ShapeDtypeStruct((B, 1, L), jnp.float32)] * 3
    return pl.pallas_call(
        _tc_body,
        grid=(B // NB,),
        in_specs=in_specs,
        out_specs=out_specs,
        out_shape=out_shape,
        interpret=interpret,
    )(se3, ts3, sk3, tg3, mk3, us3, it3, lg3,
      w1a, w1b, c8, w2t, b2r, wsc, bsr, wu, wi, wl, blinr, fe, tri, febf, i3)


def _prep(users, items, langs, skills, timestamps, targets, mask, W1, b1,
          W2, b2, Ws, bs, blin, g):
    set3 = g.reshape(B, L, DP)
    w1a = W1[:, :E].astype(jnp.bfloat16)
    w1b = W1[:, E:2 * E].astype(jnp.bfloat16)
    c8 = (jnp.concatenate(
        [W1[:, 2 * E:2 * E + 6], jnp.zeros((H1, 2), jnp.float32)], axis=1)
        + b1[:, None])
    w2 = W2.astype(jnp.bfloat16)
    wsc = Ws
    b2r = b2.reshape(H2, 1)
    bsr = bs.reshape(1, 1)
    ts3 = timestamps.astype(jnp.float32).reshape(B, 1, L)
    sk3 = skills.astype(jnp.float32).reshape(B, 1, L)
    tg3 = targets.reshape(B, 1, L)
    mk3 = jnp.asarray(mask).astype(jnp.float32).reshape(B, 1, L)
    us3 = users.reshape(B, 1, 32)
    it3 = items.reshape(B, L, 32)
    lg3 = langs.reshape(B, L, 16)
    return (set3, ts3, sk3, tg3, mk3, us3, it3, lg3,
            w1a, w1b, c8, w2, b2r, wsc, bsr)


def _wlin_split(Wlin):
    w = Wlin[0]
    wu = w[:32].reshape(1, 32)
    wi = w[32:64].reshape(1, 32)
    wl = w[64:80].reshape(1, 16)
    wk = w[80:80 + K]
    wt1 = w[80 + K:80 + 2 * K]
    wt2 = w[80 + 2 * K:80 + 3 * K]
    return wu, wi, wl, wk, wt1, wt2


def kernel(users, items, langs, skills, timestamps, targets, mask, emb_table,
           W1, b1, W2, b2, Ws, bs, Wlin, blin):
    wu, wi, wl, wk, wt1, wt2 = _wlin_split(Wlin)
    table = jnp.concatenate(
        [emb_table, wk[:, None], wt1[:, None], wt2[:, None],
         jnp.zeros((K, DP - E - 3), jnp.float32)], axis=1)
    idx = skills.reshape(-1).astype(jnp.int32).reshape(
        (B * L) // (CHUNK * NCHUNK), NCHUNK, CHUNK)
    g = _sc_gather(table, idx)
    pre = _prep(users, items, langs, skills, timestamps, targets, mask,
                W1, b1, W2, b2, Ws, bs, blin, g)
    blinr = blin.reshape(1, 1)
    fet = jnp.asarray(_FE.T.copy())
    trit = jnp.asarray(_TRI.T.copy())
    i3 = jnp.eye(3, dtype=jnp.float32)
    loss3, sig3, lab3 = _tc_call(False, *pre, wu, wi, wl, blinr,
                                 fet, trit.astype(jnp.bfloat16),
                                 fet.astype(jnp.bfloat16), i3)
    return (loss3.reshape(-1), sig3.reshape(-1), lab3.reshape(-1))


# speedup vs baseline: 1.0445x; 1.0064x over previous
import jax
import jax.numpy as jnp
import numpy as np
from jax import lax
from jax.experimental import pallas as pl
from jax.experimental.pallas import tpu as pltpu
from jax.experimental.pallas import tpu_sc as plsc

L = 50
K = 2000
E = 64
B = 128
SLOTS = 25 * L
DP = 128
CHUNK = 40
NCHUNK = 5
H1 = 256
H2 = 128
NB = 16


def _pair_tables():
    r = np.arange(25)[:, None]
    c = np.arange(L)[None, :]
    first = c < r + 1
    iof = np.where(first, r + 1, 49 - r).reshape(-1)
    jof = np.where(first, c, c - (r + 1)).reshape(-1)
    valid = ((r < 24) | (c < 25)).reshape(-1)
    slots = np.arange(SLOTS)
    fe = np.zeros((SLOTS, 2 * L), np.float32)
    fe[slots[valid], iof[valid]] = 1.0
    fe[slots[valid], L + jof[valid]] = 1.0
    tri = np.zeros((L, SLOTS), np.float32)
    tri[iof[valid], slots[valid]] = 1.0
    return fe, tri


_FE, _TRI = _pair_tables()


def _sc_gather_body(table_hbm, idx_hbm, out_hbm, idx_v, rows_v, sem):
    wid = lax.axis_index("s") * 2 + lax.axis_index("c")
    pltpu.sync_copy(idx_hbm.at[wid], idx_v)
    copies = [
        pltpu.async_copy(table_hbm.at[idx_v.at[c]],
                         rows_v.at[pl.ds(c * CHUNK, CHUNK)], sem)
        for c in range(NCHUNK)
    ]
    for cp in copies:
        cp.wait()
    pltpu.sync_copy(rows_v, out_hbm.at[pl.ds(wid * (CHUNK * NCHUNK),
                                             CHUNK * NCHUNK)])


def _sc_gather(table, idx):
    mesh = plsc.VectorSubcoreMesh(core_axis_name="c", subcore_axis_name="s")
    f = pl.kernel(
        _sc_gather_body,
        mesh=mesh,
        out_type=jax.ShapeDtypeStruct((B * L, DP), jnp.float32),
        scratch_types=[
            pltpu.VMEM((NCHUNK, CHUNK), jnp.int32),
            pltpu.VMEM((CHUNK * NCHUNK, DP), jnp.float32),
            pltpu.SemaphoreType.DMA,
        ],
    )
    return f(table, idx)


def _tc_body(se_ref, ts_ref, sk_ref, tg_ref, mk_ref, us_ref, it_ref, lg_ref,
             w1a_ref, w1b_ref, c8_ref, w2_ref, b2_ref, wsc_ref,
             bs_ref, wu_ref, wi_ref, wl_ref, blin_ref, fe_ref, tri_ref,
             febf_ref, i3_ref, loss_ref, sig_ref, lab_ref):
    f32 = jnp.float32
    bf = jnp.bfloat16

    def dott(a, b):
        return lax.dot_general(a, b, (((1,), (1,)), ((), ())),
                               preferred_element_type=f32)

    def one(ib):
        ge = se_ref[ib]
        set_ = ge[:, :E]
        pt = dott(w1a_ref[...], set_)
        qt = dott(w1b_ref[...], set_)
        g3 = dott(i3_ref[...], ge[:, E:E + 3])
        trow = ts_ref[ib]
        skrow = sk_ref[ib]
        tgrow = tg_ref[ib]
        padrow = (skrow == 0.0).astype(f32)

        zrow = jnp.zeros_like(trow)
        m3 = jnp.concatenate(
            [jnp.concatenate([trow, -trow], axis=1),
             jnp.concatenate([padrow, padrow], axis=1),
             jnp.concatenate([zrow, tgrow], axis=1)], axis=0)
        e3 = jnp.dot(m3, fe_ref[...], preferred_element_type=f32)
        dt = e3[0:1, :]
        padsum = e3[1:2, :]
        tgj = e3[2:3, :]

        cat = (1.0 + (dt > 1.0).astype(f32) + (dt > 3600.0).astype(f32)
               + (dt > 86400.0).astype(f32) + (dt > 604800.0).astype(f32))
        cat = jnp.where(padsum > 0.0, 0.0, cat)
        oh = (cat.astype(jnp.int32)
              == lax.broadcasted_iota(jnp.int32, (8, SLOTS), 0))

        aall = jnp.concatenate([pt, qt, c8_ref[...]],
                               axis=1).astype(bf)
        lhs = jnp.concatenate([febf_ref[...], oh.astype(bf)], axis=0)
        h1 = jnp.maximum(
            jnp.dot(aall, lhs, preferred_element_type=f32).astype(bf),
            0.0)
        h2 = jnp.maximum(
            jnp.dot(w2_ref[...], h1, preferred_element_type=f32)
            + b2_ref[...], 0.0)
        s = jnp.tanh(
            jnp.dot(wsc_ref[...], h2, preferred_element_type=f32)
            + bs_ref[0, 0])

        padf = 1.0 - padrow
        sv = jnp.concatenate([s, s * tgj], axis=0)
        vals = jnp.dot(sv, tri_ref[...], preferred_element_type=f32)
        vals1 = vals[0:1, :] * padf
        vals2 = vals[1:2, :] * padf

        udot = jnp.sum(us_ref[ib] * wu_ref[...])
        itdot = dott(wi_ref[...], it_ref[ib])
        ldot = dott(wl_ref[...], lg_ref[ib])
        logits = (udot + itdot + ldot + g3[0:1, :]
                  + g3[1:2, :] * vals1 + g3[2:3, :] * vals2
                  + blin_ref[0, 0])
        m = mk_ref[ib]
        preds = logits * m
        labels = tgrow * m
        loss_ref[ib] = (jnp.maximum(preds, 0.0) - preds * labels
                        + jnp.log1p(jnp.exp(-jnp.abs(preds))))
        sig_ref[ib] = 1.0 / (1.0 + jnp.exp(-preds))
        lab_ref[ib] = labels

    for ib in range(NB):
        one(ib)


def _tc_call(se3, ts3, sk3, tg3, mk3, us3, it3, lg3,
             w1a, w1b, c8, w2t, b2r, wsc, bsr, wu, wi, wl, blinr,
             fe, tri, febf, i3):
    def perb(shape):
        return pl.BlockSpec((NB,) + shape[1:], lambda i: (i, 0, 0))

    def const(arr):
        return pl.BlockSpec(arr.shape, lambda i: (0,) * arr.ndim)

    in_specs = [perb(se3.shape), perb(ts3.shape), perb(sk3.shape),
                perb(tg3.shape), perb(mk3.shape), perb(us3.shape),
                perb(it3.shape), perb(lg3.shape),
                const(w1a), const(w1b), const(c8), const(w2t),
                const(b2r), const(wsc), const(bsr), const(wu), const(wi),
                const(wl), const(blinr), const(fe), const(tri), const(febf),
                const(i3)]
    out_specs = [perb((B, 1, L))] * 3
    out_shape = [jax.ShapeDtypeStruct((B, 1, L), jnp.float32)] * 3
    return pl.pallas_call(
        _tc_body,
        grid=(B // NB,),
        in_specs=in_specs,
        out_specs=out_specs,
        out_shape=out_shape,
    )(se3, ts3, sk3, tg3, mk3, us3, it3, lg3,
      w1a, w1b, c8, w2t, b2r, wsc, bsr, wu, wi, wl, blinr, fe, tri, febf, i3)


def _prep(users, items, langs, skills, timestamps, targets, mask, W1, b1,
          W2, b2, Ws, bs, blin, g):
    set3 = g.reshape(B, L, DP)
    w1a = W1[:, :E]
    w1b = W1[:, E:2 * E]
    c8 = (jnp.concatenate(
        [W1[:, 2 * E:2 * E + 6], jnp.zeros((H1, 2), jnp.float32)], axis=1)
        + b1[:, None])
    w2 = W2.astype(jnp.bfloat16)
    wsc = Ws
    b2r = b2.reshape(H2, 1)
    bsr = bs.reshape(1, 1)
    ts3 = timestamps.astype(jnp.float32).reshape(B, 1, L)
    sk3 = skills.astype(jnp.float32).reshape(B, 1, L)
    tg3 = targets.reshape(B, 1, L)
    mk3 = jnp.asarray(mask).astype(jnp.float32).reshape(B, 1, L)
    us3 = users.reshape(B, 1, 32)
    it3 = items.reshape(B, L, 32)
    lg3 = langs.reshape(B, L, 16)
    return (set3, ts3, sk3, tg3, mk3, us3, it3, lg3,
            w1a, w1b, c8, w2, b2r, wsc, bsr)


def _wlin_split(Wlin):
    w = Wlin[0]
    wu = w[:32].reshape(1, 32)
    wi = w[32:64].reshape(1, 32)
    wl = w[64:80].reshape(1, 16)
    wk = w[80:80 + K]
    wt1 = w[80 + K:80 + 2 * K]
    wt2 = w[80 + 2 * K:80 + 3 * K]
    return wu, wi, wl, wk, wt1, wt2


def kernel(users, items, langs, skills, timestamps, targets, mask, emb_table,
           W1, b1, W2, b2, Ws, bs, Wlin, blin):
    wu, wi, wl, wk, wt1, wt2 = _wlin_split(Wlin)
    table = jnp.concatenate(
        [emb_table, wk[:, None], wt1[:, None], wt2[:, None],
         jnp.zeros((K, DP - E - 3), jnp.float32)], axis=1)
    idx = skills.reshape(-1).astype(jnp.int32).reshape(
        (B * L) // (CHUNK * NCHUNK), NCHUNK, CHUNK)
    g = _sc_gather(table, idx)
    pre = _prep(users, items, langs, skills, timestamps, targets, mask,
                W1, b1, W2, b2, Ws, bs, blin, g)
    blinr = blin.reshape(1, 1)
    fet = jnp.asarray(_FE.T.copy())
    trit = jnp.asarray(_TRI.T.copy())
    i3 = jnp.eye(3, dtype=jnp.float32)
    loss3, sig3, lab3 = _tc_call(*pre, wu, wi, wl, blinr,
                                 fet, trit, fet.astype(jnp.bfloat16), i3)
    return (loss3.reshape(-1), sig3.reshape(-1), lab3.reshape(-1))


# speedup vs baseline: 1.0512x; 1.0065x over previous
import jax
import jax.numpy as jnp
import numpy as np
from jax import lax
from jax.experimental import pallas as pl
from jax.experimental.pallas import tpu as pltpu
from jax.experimental.pallas import tpu_sc as plsc

L = 50
K = 2000
E = 64
B = 128
SLOTS = 25 * L
DP = 128
CHUNK = 40
NCHUNK = 5
H1 = 256
H2 = 128
NB = 32


def _pair_tables():
    r = np.arange(25)[:, None]
    c = np.arange(L)[None, :]
    first = c < r + 1
    iof = np.where(first, r + 1, 49 - r).reshape(-1)
    jof = np.where(first, c, c - (r + 1)).reshape(-1)
    valid = ((r < 24) | (c < 25)).reshape(-1)
    slots = np.arange(SLOTS)
    fe = np.zeros((SLOTS, 2 * L), np.float32)
    fe[slots[valid], iof[valid]] = 1.0
    fe[slots[valid], L + jof[valid]] = 1.0
    tri = np.zeros((L, SLOTS), np.float32)
    tri[iof[valid], slots[valid]] = 1.0
    return fe, tri


_FE, _TRI = _pair_tables()


def _sc_gather_body(table_hbm, idx_hbm, out_hbm, idx_v, rows_v, sem):
    wid = lax.axis_index("s") * 2 + lax.axis_index("c")
    pltpu.sync_copy(idx_hbm.at[wid], idx_v)
    copies = [
        pltpu.async_copy(table_hbm.at[idx_v.at[c]],
                         rows_v.at[pl.ds(c * CHUNK, CHUNK)], sem)
        for c in range(NCHUNK)
    ]
    for cp in copies:
        cp.wait()
    pltpu.sync_copy(rows_v, out_hbm.at[pl.ds(wid * (CHUNK * NCHUNK),
                                             CHUNK * NCHUNK)])


def _sc_gather(table, idx):
    mesh = plsc.VectorSubcoreMesh(core_axis_name="c", subcore_axis_name="s")
    f = pl.kernel(
        _sc_gather_body,
        mesh=mesh,
        out_type=jax.ShapeDtypeStruct((B * L, DP), jnp.float32),
        scratch_types=[
            pltpu.VMEM((NCHUNK, CHUNK), jnp.int32),
            pltpu.VMEM((CHUNK * NCHUNK, DP), jnp.float32),
            pltpu.SemaphoreType.DMA,
        ],
    )
    return f(table, idx)


def _tc_body(se_ref, ts_ref, sk_ref, tg_ref, mk_ref, us_ref, it_ref, lg_ref,
             w1a_ref, w1b_ref, c8_ref, w2_ref, b2_ref, wsc_ref,
             bs_ref, wu_ref, wi_ref, wl_ref, blin_ref, fe_ref, tri_ref,
             febf_ref, i3_ref, loss_ref, sig_ref, lab_ref):
    f32 = jnp.float32
    bf = jnp.bfloat16

    def dott(a, b):
        return lax.dot_general(a, b, (((1,), (1,)), ((), ())),
                               preferred_element_type=f32)

    def one(ib):
        ge = se_ref[ib]
        set_ = ge[:, :E]
        pt = dott(w1a_ref[...], set_)
        qt = dott(w1b_ref[...], set_)
        g3 = dott(i3_ref[...], ge[:, E:E + 3])
        trow = ts_ref[ib]
        skrow = sk_ref[ib]
        tgrow = tg_ref[ib]
        padrow = (skrow == 0.0).astype(f32)

        zrow = jnp.zeros_like(trow)
        m3 = jnp.concatenate(
            [jnp.concatenate([trow, -trow], axis=1),
             jnp.concatenate([padrow, padrow], axis=1),
             jnp.concatenate([zrow, tgrow], axis=1)], axis=0)
        e3 = jnp.dot(m3, fe_ref[...], preferred_element_type=f32)
        dt = e3[0:1, :]
        padsum = e3[1:2, :]
        tgj = e3[2:3, :]

        cat = (1.0 + (dt > 1.0).astype(f32) + (dt > 3600.0).astype(f32)
               + (dt > 86400.0).astype(f32) + (dt > 604800.0).astype(f32))
        cat = jnp.where(padsum > 0.0, 0.0, cat)
        oh = (cat.astype(jnp.int32)
              == lax.broadcasted_iota(jnp.int32, (8, SLOTS), 0))

        aall = jnp.concatenate([pt, qt, c8_ref[...]],
                               axis=1).astype(bf)
        lhs = jnp.concatenate([febf_ref[...], oh.astype(bf)], axis=0)
        h1 = jnp.maximum(
            jnp.dot(aall, lhs, preferred_element_type=f32).astype(bf),
            0.0)
        h2 = jnp.maximum(
            jnp.dot(w2_ref[...], h1, preferred_element_type=f32)
            + b2_ref[...], 0.0)
        s = jnp.tanh(
            jnp.dot(wsc_ref[...], h2, preferred_element_type=f32)
            + bs_ref[0, 0])

        padf = 1.0 - padrow
        sv = jnp.concatenate([s, s * tgj], axis=0)
        vals = jnp.dot(sv, tri_ref[...], preferred_element_type=f32)
        vals1 = vals[0:1, :] * padf
        vals2 = vals[1:2, :] * padf

        udot = jnp.sum(us_ref[ib] * wu_ref[...])
        itdot = dott(wi_ref[...], it_ref[ib])
        ldot = dott(wl_ref[...], lg_ref[ib])
        logits = (udot + itdot + ldot + g3[0:1, :]
                  + g3[1:2, :] * vals1 + g3[2:3, :] * vals2
                  + blin_ref[0, 0])
        m = mk_ref[ib]
        preds = logits * m
        labels = tgrow * m
        loss_ref[ib] = (jnp.maximum(preds, 0.0) - preds * labels
                        + jnp.log1p(jnp.exp(-jnp.abs(preds))))
        sig_ref[ib] = 1.0 / (1.0 + jnp.exp(-preds))
        lab_ref[ib] = labels

    for ib in range(NB):
        one(ib)


def _tc_call(se3, ts3, sk3, tg3, mk3, us3, it3, lg3,
             w1a, w1b, c8, w2t, b2r, wsc, bsr, wu, wi, wl, blinr,
             fe, tri, febf, i3):
    def perb(shape):
        return pl.BlockSpec((NB,) + shape[1:], lambda i: (i, 0, 0))

    def const(arr):
        return pl.BlockSpec(arr.shape, lambda i: (0,) * arr.ndim)

    in_specs = [perb(se3.shape), perb(ts3.shape), perb(sk3.shape),
                perb(tg3.shape), perb(mk3.shape), perb(us3.shape),
                perb(it3.shape), perb(lg3.shape),
                const(w1a), const(w1b), const(c8), const(w2t),
                const(b2r), const(wsc), const(bsr), const(wu), const(wi),
                const(wl), const(blinr), const(fe), const(tri), const(febf),
                const(i3)]
    out_specs = [perb((B, 1, L))] * 3
    out_shape = [jax.ShapeDtypeStruct((B, 1, L), jnp.float32)] * 3
    return pl.pallas_call(
        _tc_body,
        grid=(B // NB,),
        in_specs=in_specs,
        out_specs=out_specs,
        out_shape=out_shape,
    )(se3, ts3, sk3, tg3, mk3, us3, it3, lg3,
      w1a, w1b, c8, w2t, b2r, wsc, bsr, wu, wi, wl, blinr, fe, tri, febf, i3)


def _prep(users, items, langs, skills, timestamps, targets, mask, W1, b1,
          W2, b2, Ws, bs, blin, g):
    set3 = g.reshape(B, L, DP)
    w1a = W1[:, :E]
    w1b = W1[:, E:2 * E]
    c8 = (jnp.concatenate(
        [W1[:, 2 * E:2 * E + 6], jnp.zeros((H1, 2), jnp.float32)], axis=1)
        + b1[:, None])
    w2 = W2.astype(jnp.bfloat16)
    wsc = Ws
    b2r = b2.reshape(H2, 1)
    bsr = bs.reshape(1, 1)
    ts3 = timestamps.astype(jnp.float32).reshape(B, 1, L)
    sk3 = skills.astype(jnp.float32).reshape(B, 1, L)
    tg3 = targets.reshape(B, 1, L)
    mk3 = jnp.asarray(mask).astype(jnp.float32).reshape(B, 1, L)
    us3 = users.reshape(B, 1, 32)
    it3 = items.reshape(B, L, 32)
    lg3 = langs.reshape(B, L, 16)
    return (set3, ts3, sk3, tg3, mk3, us3, it3, lg3,
            w1a, w1b, c8, w2, b2r, wsc, bsr)


def _wlin_split(Wlin):
    w = Wlin[0]
    wu = w[:32].reshape(1, 32)
    wi = w[32:64].reshape(1, 32)
    wl = w[64:80].reshape(1, 16)
    wk = w[80:80 + K]
    wt1 = w[80 + K:80 + 2 * K]
    wt2 = w[80 + 2 * K:80 + 3 * K]
    return wu, wi, wl, wk, wt1, wt2


def kernel(users, items, langs, skills, timestamps, targets, mask, emb_table,
           W1, b1, W2, b2, Ws, bs, Wlin, blin):
    wu, wi, wl, wk, wt1, wt2 = _wlin_split(Wlin)
    table = jnp.concatenate(
        [emb_table, wk[:, None], wt1[:, None], wt2[:, None],
         jnp.zeros((K, DP - E - 3), jnp.float32)], axis=1)
    idx = skills.reshape(-1).astype(jnp.int32).reshape(
        (B * L) // (CHUNK * NCHUNK), NCHUNK, CHUNK)
    g = _sc_gather(table, idx)
    pre = _prep(users, items, langs, skills, timestamps, targets, mask,
                W1, b1, W2, b2, Ws, bs, blin, g)
    blinr = blin.reshape(1, 1)
    fet = jnp.asarray(_FE.T.copy())
    trit = jnp.asarray(_TRI.T.copy())
    i3 = jnp.eye(3, dtype=jnp.float32)
    loss3, sig3, lab3 = _tc_call(*pre, wu, wi, wl, blinr,
                                 fet, trit, fet.astype(jnp.bfloat16), i3)
    return (loss3.reshape(-1), sig3.reshape(-1), lab3.reshape(-1))
